# baseline (speedup 1.0000x reference)
import functools

import jax
import jax.numpy as jnp
from jax import lax
from jax.experimental import pallas as pl
from jax.experimental.pallas import tpu as pltpu
from jax.experimental.pallas import tpu_sc as plsc

N_DEP = 10000
N_OBJ = 1024
E_DEP = 320000
E_OBJ = 16384
D = 128
NH = 3

NSC = 2
NT = 16
KW = 128
JUNK = N_DEP
N_DEP_P = 10240
RPT = N_DEP_P // NT

KA = 128
W_AGG = 80
ACC_P = 10112
RPT_A = ACC_P // NT
EPC = E_DEP // (NSC * NT)
W_HIST = 160
EPH = E_DEP // NT
W_OBJ = 8

_mesh = plsc.VectorSubcoreMesh(core_axis_name="c", subcore_axis_name="s")


@functools.partial(
    pl.kernel,
    out_type=jax.ShapeDtypeStruct((N_DEP_P,), jnp.float32),
    mesh=_mesh,
    scratch_types=[
        pltpu.VMEM_SHARED((N_DEP_P,), jnp.float32),
        pltpu.VMEM((W_HIST, KW), jnp.int32),
        pltpu.VMEM((KW,), jnp.float32),
    ],
)
def _sc_hist(ddst_hbm, zeros_hbm, hist_hbm, hist_sh, ddst_vm, ones_vm):
    c = lax.axis_index("c")
    s = lax.axis_index("s")
    CH = N_DEP_P // NT

    def fill_ones(j, _):
        ones_vm[pl.ds(j * 16, 16)] = jnp.full((16,), 1.0, jnp.float32)
        return 0
    lax.fori_loop(0, KW // 16, fill_ones, 0)

    @pl.when(c == 0)
    def _():
        pltpu.sync_copy(zeros_hbm.at[pl.ds(0, CH)],
                        hist_sh.at[pl.ds(s * CH, CH)])
        pltpu.sync_copy(ddst_hbm.at[s], ddst_vm)

    plsc.subcore_barrier()

    @pl.when(c == 0)
    def _():
        def body(w, _):
            pltpu.sync_copy(ones_vm, hist_sh.at[ddst_vm.at[w]], add=True)
            return 0
        lax.fori_loop(0, W_HIST, body, 0)

    plsc.subcore_barrier()

    @pl.when(c == 0)
    def _():
        pltpu.sync_copy(hist_sh.at[pl.ds(s * CH, CH)],
                        hist_hbm.at[pl.ds(s * CH, CH)])


@functools.partial(
    pl.kernel,
    out_type=jax.ShapeDtypeStruct((N_OBJ * N_OBJ,), jnp.float32),
    mesh=_mesh,
    scratch_types=[
        pltpu.VMEM_SHARED((N_OBJ * N_OBJ,), jnp.float32),
        pltpu.VMEM((W_OBJ, KW), jnp.int32),
        pltpu.VMEM((W_OBJ, KW), jnp.int32),
        pltpu.VMEM((KW,), jnp.int32),
        pltpu.VMEM((KW,), jnp.float32),
    ],
)
def _sc_adj(osrc_hbm, odst_hbm, zeros_hbm, adj_hbm,
            adj_sh, osrc_vm, odst_vm, flat_vm, ones_vm):
    c = lax.axis_index("c")
    s = lax.axis_index("s")
    MCH = (N_OBJ * N_OBJ) // NT

    def fill_ones(j, _):
        ones_vm[pl.ds(j * 16, 16)] = jnp.full((16,), 1.0, jnp.float32)
        return 0
    lax.fori_loop(0, KW // 16, fill_ones, 0)

    @pl.when(c == 0)
    def _():
        pltpu.sync_copy(zeros_hbm.at[pl.ds(0, MCH)],
                        adj_sh.at[pl.ds(s * MCH, MCH)])
        pltpu.sync_copy(osrc_hbm.at[s], osrc_vm)
        pltpu.sync_copy(odst_hbm.at[s], odst_vm)

    plsc.subcore_barrier()

    @pl.when(c == 0)
    def _():
        def body(w, _):
            def pack(j, _):
                sv = osrc_vm[w, pl.ds(j * 16, 16)]
                dv = odst_vm[w, pl.ds(j * 16, 16)]
                flat_vm[pl.ds(j * 16, 16)] = dv * N_OBJ + sv
                return 0
            lax.fori_loop(0, KW // 16, pack, 0)
            pltpu.sync_copy(ones_vm, adj_sh.at[flat_vm], add=True)
            return 0
        lax.fori_loop(0, W_OBJ, body, 0)

    plsc.subcore_barrier()

    @pl.when(c == 0)
    def _():
        pltpu.sync_copy(adj_sh.at[pl.ds(s * MCH, MCH)],
                        adj_hbm.at[pl.ds(s * MCH, MCH)])


@functools.partial(
    pl.kernel,
    out_type=jax.ShapeDtypeStruct((NSC, ACC_P, D), jnp.float32),
    mesh=_mesh,
    scratch_types=[
        pltpu.VMEM_SHARED((ACC_P, D), jnp.float32),
        pltpu.VMEM((W_AGG, KA), jnp.int32),
        pltpu.VMEM((2, KA), jnp.int32),
        pltpu.VMEM((2, KA), jnp.int32),
        pltpu.VMEM((2, KA, D), jnp.float32),
        pltpu.SemaphoreType.DMA,
        pltpu.SemaphoreType.DMA,
    ],
)
def _sc_aggregate(hp_hbm, zeros_hbm, packed_hbm, acc_hbm,
                  acc_sh, packed_vm, src_idx, dst_idx, rows_vm, sem0, sem1):
    c = lax.axis_index("c")
    s = lax.axis_index("s")
    r0 = s * RPT_A
    pltpu.sync_copy(zeros_hbm, acc_sh.at[pl.ds(r0, RPT_A)])
    pltpu.sync_copy(packed_hbm.at[c, s], packed_vm)
    plsc.subcore_barrier()

    def unpack(w, slot):
        def up(j, _):
            pv = packed_vm[w, pl.ds(j * 16, 16)]
            src_idx[slot, pl.ds(j * 16, 16)] = pv & 0xFFFF
            dst_idx[slot, pl.ds(j * 16, 16)] = lax.shift_right_logical(pv, 16)
            return 0
        lax.fori_loop(0, KA // 16, up, 0)

    def gstart(w, slot, gs):
        pltpu.async_copy(hp_hbm.at[src_idx.at[slot]], rows_vm.at[slot], gs)

    def gwait(slot, gs):
        pltpu.make_async_copy(zeros_hbm.at[pl.ds(0, KA)], rows_vm.at[slot],
                              gs).wait()

    unpack(0, 0)
    gstart(0, 0, sem0)

    def body(g, _):
        w0 = 2 * g
        w1 = w0 + 1
        gwait(0, sem0)
        unpack(w1, 1)
        gstart(w1, 1, sem1)
        pltpu.sync_copy(rows_vm.at[0], acc_sh.at[dst_idx.at[0]], add=True)
        gwait(1, sem1)

        @pl.when(w0 + 2 < W_AGG)
        def _():
            unpack(w0 + 2, 0)
            gstart(w0 + 2, 0, sem0)
        pltpu.sync_copy(rows_vm.at[1], acc_sh.at[dst_idx.at[1]], add=True)
        return 0
    lax.fori_loop(0, W_AGG // 2, body, 0)

    plsc.subcore_barrier()
    pltpu.sync_copy(acc_sh.at[pl.ds(r0, RPT_A)], acc_hbm.at[c, pl.ds(r0, RPT_A)])


def _obj_front_body(labels_ref, boxes_ref, ws_ref, wo_ref, wbs_ref, wbo_ref,
                    wo0_ref, st_ref, h0_ref):
    labels = labels_ref[...]
    ids = lax.broadcasted_iota(jnp.int32, (N_OBJ, 1600), 1)
    onehot = (ids == labels).astype(jnp.bfloat16)
    f32 = jnp.float32

    def ohdot(w):
        return lax.dot_general(onehot, w.astype(jnp.bfloat16),
                               (((1,), (0,)), ((), ())),
                               preferred_element_type=f32)
    subj = ohdot(ws_ref[...]) + boxes_ref[...] @ wbs_ref[...]
    objf = ohdot(wo_ref[...]) + boxes_ref[...] @ wbo_ref[...]
    st_ref[...] = lax.dot_general(objf, subj, (((1,), (1,)), ((), ())))
    h0_ref[...] = onehot.astype(f32) @ wo0_ref[...]


def _obj_conv_body(adj_ref, st_ref, h0_ref, bo0_ref, wq_ref, wvo_ref,
                   objh_ref, q_ref, vo_ref):
    sig = 1.0 / (1.0 + jnp.exp(-st_ref[...]))
    a = adj_ref[...] * sig
    deg = jnp.sum(a, axis=1, keepdims=True) + 1.0
    dinv = lax.rsqrt(jnp.maximum(deg, 1e-12))
    hp = dinv * h0_ref[...]
    out = dinv * (a @ hp + hp) + bo0_ref[...]
    objh_ref[...] = out
    for h in range(NH):
        q_ref[h] = out @ wq_ref[h]
        vo_ref[h] = out @ wvo_ref[h]


def _dep_prep0_body(x_ref, w_ref, hist_ref, dinv_ref, hp_ref):
    deg = hist_ref[...] + 1.0
    dinv = lax.rsqrt(jnp.maximum(deg, 1e-12))
    dinv_ref[...] = dinv
    hp_ref[...] = dinv * (x_ref[...] @ w_ref[...])


def _dep_prep1_body(x_ref, w_ref, dinv_ref, hp_ref):
    hp_ref[...] = dinv_ref[...] * (x_ref[...] @ w_ref[...])


def _dep_prep2_body(acca_ref, accb_ref, hp1_ref, dinv_ref, bt1_ref, w_ref,
                    hp_ref):
    dinv = dinv_ref[...]
    dep1 = dinv * (acca_ref[...] + accb_ref[...] + hp1_ref[...]) + bt1_ref[...]
    hp_ref[...] = dinv * (dep1 @ w_ref[...])


BT = 400
BGRID = N_DEP // BT


def _biatt_body(acca_ref, accb_ref, hp0_ref, dinv_ref, bt0_ref, wk_ref,
                wvd_ref, q_ref, vo_ref, depo_ref, objo_ref,
                cmax_ref, csum_ref, cacc_ref):
    i = pl.program_id(0)
    scale = 1.0 / jnp.sqrt(jnp.float32(D))
    dinv = dinv_ref[...]
    dep_in = dinv * (acca_ref[...] + accb_ref[...] + hp0_ref[...]) \
        + bt0_ref[...]

    @pl.when(i == 0)
    def _():
        cmax_ref[...] = jnp.full_like(cmax_ref, -1e30)
        csum_ref[...] = jnp.zeros_like(csum_ref)
        cacc_ref[...] = jnp.zeros_like(cacc_ref)

    dep_acc = jnp.zeros((BT, D), jnp.float32)
    for h in range(NH):
        k_t = dep_in @ wk_ref[h]
        vd_t = dep_in @ wvd_ref[h]
        a = lax.dot_general(k_t, q_ref[h],
                            (((1,), (1,)), ((), ()))) * scale
        rmax = jnp.max(a, axis=1, keepdims=True)
        p = jnp.exp(a - rmax)
        rsum = jnp.sum(p, axis=1, keepdims=True)
        pn = (p / rsum).astype(jnp.bfloat16)
        dep_acc = dep_acc + lax.dot_general(
            pn, vo_ref[h].astype(jnp.bfloat16), (((1,), (0,)), ((), ())),
            preferred_element_type=jnp.float32)
        tmax = jnp.max(a, axis=0, keepdims=True)
        old_m = cmax_ref[h]
        new_m = jnp.maximum(old_m, tmax)
        corr = jnp.exp(old_m - new_m)
        e = jnp.exp(a - new_m)
        cmax_ref[h] = new_m
        csum_ref[h] = csum_ref[h] * corr + jnp.sum(e, axis=0, keepdims=True)
        corr_t = corr.reshape(N_OBJ, 1)
        cacc_ref[h] = cacc_ref[h] * corr_t + lax.dot_general(
            e.astype(jnp.bfloat16), vd_t.astype(jnp.bfloat16),
            (((0,), (0,)), ((), ())),
            preferred_element_type=jnp.float32)

    depo_ref[...] = dep_acc * (1.0 / NH)

    @pl.when(i == BGRID - 1)
    def _():
        acc = jnp.zeros((N_OBJ, D), jnp.float32)
        for h in range(NH):
            acc = acc + cacc_ref[h] / csum_ref[h].reshape(N_OBJ, 1)
        objo_ref[...] = acc * (1.0 / NH)


def _final_body(acca_ref, accb_ref, hp2_ref, dinv_ref, bt2_ref, objh_ref,
                dbatch_ref, obatch_ref, f1_ref, fb1_ref, f2_ref, fb2_ref,
                out_ref):
    dinv = dinv_ref[...]
    dep2 = dinv * (acca_ref[...] + accb_ref[...] + hp2_ref[...]) \
        + bt2_ref[...]
    db = dbatch_ref[...]
    ids = lax.broadcasted_iota(jnp.int32, (N_DEP, 64), 1)
    ohd = (ids == db).astype(jnp.float32)
    dsum = lax.dot_general(ohd, dep2, (((0,), (0,)), ((), ())))
    dcnt = jnp.sum(ohd, axis=0, keepdims=True).reshape(64, 1)
    dep_p = dsum / jnp.maximum(dcnt, 1.0)

    ob = obatch_ref[...]
    ids_o = lax.broadcasted_iota(jnp.int32, (N_OBJ, 64), 1)
    oho = (ids_o == ob).astype(jnp.float32)
    osum = lax.dot_general(oho, objh_ref[...], (((0,), (0,)), ((), ())))
    ocnt = jnp.sum(oho, axis=0, keepdims=True).reshape(64, 1)
    obj_p = osum / jnp.maximum(ocnt, 1.0)

    fused = jnp.concatenate([dep_p, obj_p], axis=1)
    hmid = fused @ f1_ref[...] + fb1_ref[...]
    logits = hmid @ f2_ref[...] + fb2_ref[...]
    lmax = jnp.max(logits, axis=1, keepdims=True)
    lse = jnp.log(jnp.sum(jnp.exp(logits - lmax), axis=1, keepdims=True)) + lmax
    out_ref[...] = logits - lse


def _full_spec(shape):
    return pl.BlockSpec(shape, lambda *_: tuple(0 for _ in shape))


def kernel(dep_x, dep_edge_index, dep_batch, obj_boxes, obj_labels,
           obj_edge_index, obj_batch, Wt0, bt0, Wo0, bo0, Wk, Wq, Wvd, Wvo,
           Wt1, bt1, Wt2, bt2, Ws_rel, Wo_rel, Wbs, Wbo, F1, fb1, F2, fb2):
    f32 = jnp.float32
    i32 = jnp.int32

    padc = W_AGG * KA - EPC
    dsrc_a = dep_edge_index[0].astype(i32).reshape(NSC * NT, EPC)
    dsrc_a = jnp.concatenate(
        [dsrc_a, jnp.zeros((NSC * NT, padc), i32)], axis=1)
    ddst_a = dep_edge_index[1].astype(i32).reshape(NSC * NT, EPC)
    ddst_a = jnp.concatenate(
        [ddst_a, jnp.full((NSC * NT, padc), JUNK, i32)], axis=1)
    packed_a = ((ddst_a << 16) | dsrc_a).reshape(NSC, NT, W_AGG, KA)
    padh = W_HIST * KW - EPH
    ddst_h = dep_edge_index[1].astype(i32).reshape(NT, EPH)
    ddst_h = jnp.concatenate(
        [ddst_h, jnp.full((NT, padh), JUNK, i32)], axis=1)
    ddst_h = ddst_h.reshape(NT, W_HIST, KW)
    osrc = obj_edge_index[0].astype(i32).reshape(NT, W_OBJ, KW)
    odst = obj_edge_index[1].astype(i32).reshape(NT, W_OBJ, KW)
    zeros_row = jnp.zeros((RPT_A, D), f32)
    zeros_hist = jnp.zeros((N_DEP_P // NT,), f32)
    zeros_adj = jnp.zeros(((N_OBJ * N_OBJ) // NT,), f32)

    hist = _sc_hist(ddst_h, zeros_hist)
    adj = _sc_adj(osrc, odst, zeros_adj).reshape(N_OBJ, N_OBJ)
    hist2 = hist.reshape(N_DEP_P, 1)

    st, obj_h0 = pl.pallas_call(
        _obj_front_body,
        out_shape=(jax.ShapeDtypeStruct((N_OBJ, N_OBJ), f32),
                   jax.ShapeDtypeStruct((N_OBJ, D), f32)),
        in_specs=[_full_spec((N_OBJ, 1)), _full_spec((N_OBJ, 4)),
                  _full_spec((1600, 64)), _full_spec((1600, 64)),
                  _full_spec((4, 64)), _full_spec((4, 64)),
                  _full_spec((1600, D))],
        out_specs=(_full_spec((N_OBJ, N_OBJ)), _full_spec((N_OBJ, D))),
    )(obj_labels.astype(i32).reshape(N_OBJ, 1), obj_boxes, Ws_rel, Wo_rel,
      Wbs, Wbo, Wo0)

    obj_h, q, vo = pl.pallas_call(
        _obj_conv_body,
        out_shape=(jax.ShapeDtypeStruct((N_OBJ, D), f32),
                   jax.ShapeDtypeStruct((NH, N_OBJ, D), f32),
                   jax.ShapeDtypeStruct((NH, N_OBJ, D), f32)),
        in_specs=[_full_spec((N_OBJ, N_OBJ)), _full_spec((N_OBJ, N_OBJ)),
                  _full_spec((N_OBJ, D)), _full_spec((1, D)),
                  _full_spec((NH, D, D)), _full_spec((NH, D, D))],
        out_specs=(_full_spec((N_OBJ, D)), _full_spec((NH, N_OBJ, D)),
                   _full_spec((NH, N_OBJ, D))),
    )(adj, st, obj_h0, bo0.reshape(1, D), Wq, Wvo)

    BR = 2048
    dinv, hp0 = pl.pallas_call(
        _dep_prep0_body,
        grid=(N_DEP_P // BR,),
        out_shape=(jax.ShapeDtypeStruct((N_DEP_P, 1), f32),
                   jax.ShapeDtypeStruct((N_DEP_P, D), f32)),
        in_specs=[pl.BlockSpec((BR, D), lambda i: (i, 0)),
                  pl.BlockSpec((D, D), lambda i: (0, 0)),
                  pl.BlockSpec((BR, 1), lambda i: (i, 0))],
        out_specs=(pl.BlockSpec((BR, 1), lambda i: (i, 0)),
                   pl.BlockSpec((BR, D), lambda i: (i, 0))),
    )(dep_x, Wt0, hist2)

    acc0 = _sc_aggregate(hp0, zeros_row, packed_a)

    dep_hb, obj_hb = pl.pallas_call(
        _biatt_body,
        grid=(BGRID,),
        out_shape=(jax.ShapeDtypeStruct((N_DEP, D), f32),
                   jax.ShapeDtypeStruct((N_OBJ, D), f32)),
        in_specs=[pl.BlockSpec((BT, D), lambda i: (i, 0)),
                  pl.BlockSpec((BT, D), lambda i: (i, 0)),
                  pl.BlockSpec((BT, D), lambda i: (i, 0)),
                  pl.BlockSpec((BT, 1), lambda i: (i, 0)),
                  pl.BlockSpec((1, D), lambda i: (0, 0)),
                  pl.BlockSpec((NH, D, D), lambda i: (0, 0, 0)),
                  pl.BlockSpec((NH, D, D), lambda i: (0, 0, 0)),
                  pl.BlockSpec((NH, N_OBJ, D), lambda i: (0, 0, 0)),
                  pl.BlockSpec((NH, N_OBJ, D), lambda i: (0, 0, 0))],
        out_specs=(pl.BlockSpec((BT, D), lambda i: (i, 0)),
                   pl.BlockSpec((N_OBJ, D), lambda i: (0, 0))),
        scratch_shapes=[pltpu.VMEM((NH, 1, N_OBJ), f32),
                        pltpu.VMEM((NH, 1, N_OBJ), f32),
                        pltpu.VMEM((NH, N_OBJ, D), f32)],
    )(acc0[0], acc0[1], hp0, dinv, bt0.reshape(1, D), Wk, Wvd, q, vo)

    hp1 = pl.pallas_call(
        _dep_prep1_body,
        grid=(N_DEP_P // BR,),
        out_shape=jax.ShapeDtypeStruct((N_DEP_P, D), f32),
        in_specs=[pl.BlockSpec((BR, D), lambda i: (i, 0)),
                  pl.BlockSpec((D, D), lambda i: (0, 0)),
                  pl.BlockSpec((BR, 1), lambda i: (i, 0))],
        out_specs=pl.BlockSpec((BR, D), lambda i: (i, 0)),
    )(dep_hb, Wt1, dinv)

    acc1 = _sc_aggregate(hp1, zeros_row, packed_a)

    hp2 = pl.pallas_call(
        _dep_prep2_body,
        grid=(N_DEP_P // BR,),
        out_shape=jax.ShapeDtypeStruct((N_DEP_P, D), f32),
        in_specs=[pl.BlockSpec((BR, D), lambda i: (i, 0)),
                  pl.BlockSpec((BR, D), lambda i: (i, 0)),
                  pl.BlockSpec((BR, D), lambda i: (i, 0)),
                  pl.BlockSpec((BR, 1), lambda i: (i, 0)),
                  pl.BlockSpec((1, D), lambda i: (0, 0)),
                  pl.BlockSpec((D, D), lambda i: (0, 0))],
        out_specs=pl.BlockSpec((BR, D), lambda i: (i, 0)),
    )(acc1[0], acc1[1], hp1, dinv, bt1.reshape(1, D), Wt2)

    acc2 = _sc_aggregate(hp2, zeros_row, packed_a)

    hid = F1.shape[1]
    a_cls = F2.shape[1]
    out = pl.pallas_call(
        _final_body,
        out_shape=jax.ShapeDtypeStruct((64, a_cls), f32),
        in_specs=[_full_spec((N_DEP, D)), _full_spec((N_DEP, D)),
                  _full_spec((N_DEP, D)), _full_spec((N_DEP, 1)),
                  _full_spec((1, D)), _full_spec((N_OBJ, D)),
                  _full_spec((N_DEP, 1)), _full_spec((N_OBJ, 1)),
                  _full_spec((2 * D, hid)), _full_spec((1, hid)),
                  _full_spec((hid, a_cls)), _full_spec((1, a_cls))],
        out_specs=_full_spec((64, a_cls)),
        compiler_params=pltpu.CompilerParams(
            vmem_limit_bytes=100 * 1024 * 1024),
    )(acc2[0, :N_DEP], acc2[1, :N_DEP], hp2[:N_DEP], dinv[:N_DEP],
      bt2.reshape(1, D), obj_hb,
      dep_batch.astype(i32).reshape(N_DEP, 1),
      obj_batch.astype(i32).reshape(N_OBJ, 1),
      F1, fb1.reshape(1, hid), F2, fb2.reshape(1, a_cls))
    return out


# speedup vs baseline: 1.2869x; 1.2869x over previous
import functools

import jax
import jax.numpy as jnp
from jax import lax
from jax.experimental import pallas as pl
from jax.experimental.pallas import tpu as pltpu
from jax.experimental.pallas import tpu_sc as plsc

N_DEP = 10000
N_OBJ = 1024
E_DEP = 320000
E_OBJ = 16384
D = 128
NH = 3

NSC = 2
NT = 16
KW = 128
JUNK = N_DEP
N_DEP_P = 10240
RPT = N_DEP_P // NT

W_AGG = 79
EPC = E_DEP // (NSC * NT)
W_HIST = 160
EPH = E_DEP // NT
W_OBJ = 8

_mesh = plsc.VectorSubcoreMesh(core_axis_name="c", subcore_axis_name="s")


@functools.partial(
    pl.kernel,
    out_type=jax.ShapeDtypeStruct((N_DEP_P,), jnp.float32),
    mesh=_mesh,
    scratch_types=[
        pltpu.VMEM_SHARED((N_DEP_P,), jnp.float32),
        pltpu.VMEM((W_HIST, KW), jnp.int32),
        pltpu.VMEM((KW,), jnp.float32),
    ],
)
def _sc_hist(ddst_hbm, zeros_hbm, hist_hbm, hist_sh, ddst_vm, ones_vm):
    c = lax.axis_index("c")
    s = lax.axis_index("s")
    CH = N_DEP_P // NT

    def fill_ones(j, _):
        ones_vm[pl.ds(j * 16, 16)] = jnp.full((16,), 1.0, jnp.float32)
        return 0
    lax.fori_loop(0, KW // 16, fill_ones, 0)

    @pl.when(c == 0)
    def _():
        pltpu.sync_copy(zeros_hbm.at[pl.ds(0, CH)],
                        hist_sh.at[pl.ds(s * CH, CH)])
        pltpu.sync_copy(ddst_hbm.at[s], ddst_vm)

    plsc.subcore_barrier()

    @pl.when(c == 0)
    def _():
        def body(w, _):
            pltpu.sync_copy(ones_vm, hist_sh.at[ddst_vm.at[w]], add=True)
            return 0
        lax.fori_loop(0, W_HIST, body, 0)

    plsc.subcore_barrier()

    @pl.when(c == 0)
    def _():
        pltpu.sync_copy(hist_sh.at[pl.ds(s * CH, CH)],
                        hist_hbm.at[pl.ds(s * CH, CH)])


@functools.partial(
    pl.kernel,
    out_type=jax.ShapeDtypeStruct((N_OBJ * N_OBJ,), jnp.float32),
    mesh=_mesh,
    scratch_types=[
        pltpu.VMEM_SHARED((N_OBJ * N_OBJ,), jnp.float32),
        pltpu.VMEM((W_OBJ, KW), jnp.int32),
        pltpu.VMEM((W_OBJ, KW), jnp.int32),
        pltpu.VMEM((KW,), jnp.int32),
        pltpu.VMEM((KW,), jnp.float32),
    ],
)
def _sc_adj(osrc_hbm, odst_hbm, zeros_hbm, adj_hbm,
            adj_sh, osrc_vm, odst_vm, flat_vm, ones_vm):
    c = lax.axis_index("c")
    s = lax.axis_index("s")
    MCH = (N_OBJ * N_OBJ) // NT

    def fill_ones(j, _):
        ones_vm[pl.ds(j * 16, 16)] = jnp.full((16,), 1.0, jnp.float32)
        return 0
    lax.fori_loop(0, KW // 16, fill_ones, 0)

    @pl.when(c == 0)
    def _():
        pltpu.sync_copy(zeros_hbm.at[pl.ds(0, MCH)],
                        adj_sh.at[pl.ds(s * MCH, MCH)])
        pltpu.sync_copy(osrc_hbm.at[s], osrc_vm)
        pltpu.sync_copy(odst_hbm.at[s], odst_vm)

    plsc.subcore_barrier()

    @pl.when(c == 0)
    def _():
        def body(w, _):
            def pack(j, _):
                sv = osrc_vm[w, pl.ds(j * 16, 16)]
                dv = odst_vm[w, pl.ds(j * 16, 16)]
                flat_vm[pl.ds(j * 16, 16)] = dv * N_OBJ + sv
                return 0
            lax.fori_loop(0, KW // 16, pack, 0)
            pltpu.sync_copy(ones_vm, adj_sh.at[flat_vm], add=True)
            return 0
        lax.fori_loop(0, W_OBJ, body, 0)

    plsc.subcore_barrier()

    @pl.when(c == 0)
    def _():
        pltpu.sync_copy(adj_sh.at[pl.ds(s * MCH, MCH)],
                        adj_hbm.at[pl.ds(s * MCH, MCH)])


@functools.partial(
    pl.kernel,
    out_type=jax.ShapeDtypeStruct((NSC, N_DEP_P, D), jnp.float32),
    mesh=_mesh,
    scratch_types=[
        pltpu.VMEM_SHARED((N_DEP_P, D), jnp.float32),
        pltpu.VMEM((W_AGG, KW), jnp.int32),
        pltpu.VMEM((W_AGG, KW), jnp.int32),
        pltpu.VMEM((KW, D), jnp.float32),
        pltpu.SemaphoreType.DMA,
    ],
)
def _sc_aggregate(hp_hbm, zeros_hbm, src_hbm, dst_hbm, acc_hbm,
                  acc_sh, src_vm, dst_vm, rows_vm, sem):
    c = lax.axis_index("c")
    s = lax.axis_index("s")
    r0 = s * RPT
    pltpu.sync_copy(zeros_hbm, acc_sh.at[pl.ds(r0, RPT)])
    pltpu.sync_copy(src_hbm.at[c, s], src_vm)
    pltpu.sync_copy(dst_hbm.at[c, s], dst_vm)
    plsc.subcore_barrier()

    def body(w, _):
        pltpu.async_copy(hp_hbm.at[src_vm.at[w]], rows_vm, sem).wait()
        pltpu.sync_copy(rows_vm, acc_sh.at[dst_vm.at[w]], add=True)
        return 0
    lax.fori_loop(0, W_AGG, body, 0)

    plsc.subcore_barrier()
    pltpu.sync_copy(acc_sh.at[pl.ds(r0, RPT)], acc_hbm.at[c, pl.ds(r0, RPT)])


def _obj_front_body(labels_ref, boxes_ref, ws_ref, wo_ref, wbs_ref, wbo_ref,
                    wo0_ref, st_ref, h0_ref):
    labels = labels_ref[...]
    ids = lax.broadcasted_iota(jnp.int32, (N_OBJ, 1600), 1)
    onehot = (ids == labels).astype(jnp.bfloat16)
    f32 = jnp.float32

    def ohdot(w):
        return lax.dot_general(onehot, w.astype(jnp.bfloat16),
                               (((1,), (0,)), ((), ())),
                               preferred_element_type=f32)
    subj = ohdot(ws_ref[...]) + boxes_ref[...] @ wbs_ref[...]
    objf = ohdot(wo_ref[...]) + boxes_ref[...] @ wbo_ref[...]
    st_ref[...] = lax.dot_general(objf, subj, (((1,), (1,)), ((), ())))
    h0_ref[...] = onehot.astype(f32) @ wo0_ref[...]


def _obj_conv_body(adj_ref, st_ref, h0_ref, bo0_ref, wq_ref, wvo_ref,
                   objh_ref, q_ref, vo_ref):
    sig = 1.0 / (1.0 + jnp.exp(-st_ref[...]))
    a = adj_ref[...] * sig
    deg = jnp.sum(a, axis=1, keepdims=True) + 1.0
    dinv = lax.rsqrt(jnp.maximum(deg, 1e-12))
    hp = dinv * h0_ref[...]
    out = dinv * (a @ hp + hp) + bo0_ref[...]
    objh_ref[...] = out
    for h in range(NH):
        q_ref[h] = out @ wq_ref[h]
        vo_ref[h] = out @ wvo_ref[h]


def _dep_prep0_body(x_ref, w_ref, hist_ref, dinv_ref, hp_ref):
    deg = hist_ref[...] + 1.0
    dinv = lax.rsqrt(jnp.maximum(deg, 1e-12))
    dinv_ref[...] = dinv
    hp_ref[...] = dinv * (x_ref[...] @ w_ref[...])


def _dep_prep1_body(x_ref, w_ref, dinv_ref, hp_ref):
    hp_ref[...] = dinv_ref[...] * (x_ref[...] @ w_ref[...])


def _dep_prep2_body(acca_ref, accb_ref, hp1_ref, dinv_ref, bt1_ref, w_ref,
                    hp_ref):
    dinv = dinv_ref[...]
    dep1 = dinv * (acca_ref[...] + accb_ref[...] + hp1_ref[...]) + bt1_ref[...]
    hp_ref[...] = dinv * (dep1 @ w_ref[...])


BT = 400
BGRID = N_DEP // BT


def _biatt_body(acca_ref, accb_ref, hp0_ref, dinv_ref, bt0_ref, wk_ref,
                wvd_ref, q_ref, vo_ref, depo_ref, objo_ref,
                cmax_ref, csum_ref, cacc_ref):
    i = pl.program_id(0)
    scale = 1.0 / jnp.sqrt(jnp.float32(D))
    dinv = dinv_ref[...]
    dep_in = dinv * (acca_ref[...] + accb_ref[...] + hp0_ref[...]) \
        + bt0_ref[...]

    @pl.when(i == 0)
    def _():
        cmax_ref[...] = jnp.full_like(cmax_ref, -1e30)
        csum_ref[...] = jnp.zeros_like(csum_ref)
        cacc_ref[...] = jnp.zeros_like(cacc_ref)

    dep_acc = jnp.zeros((BT, D), jnp.float32)
    for h in range(NH):
        k_t = dep_in @ wk_ref[h]
        vd_t = dep_in @ wvd_ref[h]
        a = lax.dot_general(k_t, q_ref[h],
                            (((1,), (1,)), ((), ()))) * scale
        rmax = jnp.max(a, axis=1, keepdims=True)
        p = jnp.exp(a - rmax)
        rsum = jnp.sum(p, axis=1, keepdims=True)
        pn = (p / rsum).astype(jnp.bfloat16)
        dep_acc = dep_acc + lax.dot_general(
            pn, vo_ref[h].astype(jnp.bfloat16), (((1,), (0,)), ((), ())),
            preferred_element_type=jnp.float32)
        tmax = jnp.max(a, axis=0, keepdims=True)
        old_m = cmax_ref[h]
        new_m = jnp.maximum(old_m, tmax)
        corr = jnp.exp(old_m - new_m)
        e = jnp.exp(a - new_m)
        cmax_ref[h] = new_m
        csum_ref[h] = csum_ref[h] * corr + jnp.sum(e, axis=0, keepdims=True)
        corr_t = corr.reshape(N_OBJ, 1)
        cacc_ref[h] = cacc_ref[h] * corr_t + lax.dot_general(
            e.astype(jnp.bfloat16), vd_t.astype(jnp.bfloat16),
            (((0,), (0,)), ((), ())),
            preferred_element_type=jnp.float32)

    depo_ref[...] = dep_acc * (1.0 / NH)

    @pl.when(i == BGRID - 1)
    def _():
        acc = jnp.zeros((N_OBJ, D), jnp.float32)
        for h in range(NH):
            acc = acc + cacc_ref[h] / csum_ref[h].reshape(N_OBJ, 1)
        objo_ref[...] = acc * (1.0 / NH)


def _final_body(acca_ref, accb_ref, hp2_ref, dinv_ref, bt2_ref, objh_ref,
                dbatch_ref, obatch_ref, f1_ref, fb1_ref, f2_ref, fb2_ref,
                out_ref):
    dinv = dinv_ref[...]
    dep2 = dinv * (acca_ref[...] + accb_ref[...] + hp2_ref[...]) \
        + bt2_ref[...]
    db = dbatch_ref[...]
    ids = lax.broadcasted_iota(jnp.int32, (N_DEP, 64), 1)
    ohd = (ids == db).astype(jnp.float32)
    dsum = lax.dot_general(ohd, dep2, (((0,), (0,)), ((), ())))
    dcnt = jnp.sum(ohd, axis=0, keepdims=True).reshape(64, 1)
    dep_p = dsum / jnp.maximum(dcnt, 1.0)

    ob = obatch_ref[...]
    ids_o = lax.broadcasted_iota(jnp.int32, (N_OBJ, 64), 1)
    oho = (ids_o == ob).astype(jnp.float32)
    osum = lax.dot_general(oho, objh_ref[...], (((0,), (0,)), ((), ())))
    ocnt = jnp.sum(oho, axis=0, keepdims=True).reshape(64, 1)
    obj_p = osum / jnp.maximum(ocnt, 1.0)

    fused = jnp.concatenate([dep_p, obj_p], axis=1)
    hmid = fused @ f1_ref[...] + fb1_ref[...]
    logits = hmid @ f2_ref[...] + fb2_ref[...]
    lmax = jnp.max(logits, axis=1, keepdims=True)
    lse = jnp.log(jnp.sum(jnp.exp(logits - lmax), axis=1, keepdims=True)) + lmax
    out_ref[...] = logits - lse


def _full_spec(shape):
    return pl.BlockSpec(shape, lambda *_: tuple(0 for _ in shape))


def kernel(dep_x, dep_edge_index, dep_batch, obj_boxes, obj_labels,
           obj_edge_index, obj_batch, Wt0, bt0, Wo0, bo0, Wk, Wq, Wvd, Wvo,
           Wt1, bt1, Wt2, bt2, Ws_rel, Wo_rel, Wbs, Wbo, F1, fb1, F2, fb2):
    f32 = jnp.float32
    i32 = jnp.int32

    padc = W_AGG * KW - EPC
    dsrc_a = dep_edge_index[0].astype(i32).reshape(NSC * NT, EPC)
    dsrc_a = jnp.concatenate(
        [dsrc_a, jnp.zeros((NSC * NT, padc), i32)], axis=1)
    dsrc_a = dsrc_a.reshape(NSC, NT, W_AGG, KW)
    ddst_a = dep_edge_index[1].astype(i32).reshape(NSC * NT, EPC)
    ddst_a = jnp.concatenate(
        [ddst_a, jnp.full((NSC * NT, padc), JUNK, i32)], axis=1)
    ddst_a = ddst_a.reshape(NSC, NT, W_AGG, KW)
    padh = W_HIST * KW - EPH
    ddst_h = dep_edge_index[1].astype(i32).reshape(NT, EPH)
    ddst_h = jnp.concatenate(
        [ddst_h, jnp.full((NT, padh), JUNK, i32)], axis=1)
    ddst_h = ddst_h.reshape(NT, W_HIST, KW)
    osrc = obj_edge_index[0].astype(i32).reshape(NT, W_OBJ, KW)
    odst = obj_edge_index[1].astype(i32).reshape(NT, W_OBJ, KW)
    zeros_row = jnp.zeros((RPT, D), f32)
    zeros_hist = jnp.zeros((N_DEP_P // NT,), f32)
    zeros_adj = jnp.zeros(((N_OBJ * N_OBJ) // NT,), f32)

    hist = _sc_hist(ddst_h, zeros_hist)
    adj = _sc_adj(osrc, odst, zeros_adj).reshape(N_OBJ, N_OBJ)
    hist2 = hist.reshape(N_DEP_P, 1)

    st, obj_h0 = pl.pallas_call(
        _obj_front_body,
        out_shape=(jax.ShapeDtypeStruct((N_OBJ, N_OBJ), f32),
                   jax.ShapeDtypeStruct((N_OBJ, D), f32)),
        in_specs=[_full_spec((N_OBJ, 1)), _full_spec((N_OBJ, 4)),
                  _full_spec((1600, 64)), _full_spec((1600, 64)),
                  _full_spec((4, 64)), _full_spec((4, 64)),
                  _full_spec((1600, D))],
        out_specs=(_full_spec((N_OBJ, N_OBJ)), _full_spec((N_OBJ, D))),
    )(obj_labels.astype(i32).reshape(N_OBJ, 1), obj_boxes, Ws_rel, Wo_rel,
      Wbs, Wbo, Wo0)

    obj_h, q, vo = pl.pallas_call(
        _obj_conv_body,
        out_shape=(jax.ShapeDtypeStruct((N_OBJ, D), f32),
                   jax.ShapeDtypeStruct((NH, N_OBJ, D), f32),
                   jax.ShapeDtypeStruct((NH, N_OBJ, D), f32)),
        in_specs=[_full_spec((N_OBJ, N_OBJ)), _full_spec((N_OBJ, N_OBJ)),
                  _full_spec((N_OBJ, D)), _full_spec((1, D)),
                  _full_spec((NH, D, D)), _full_spec((NH, D, D))],
        out_specs=(_full_spec((N_OBJ, D)), _full_spec((NH, N_OBJ, D)),
                   _full_spec((NH, N_OBJ, D))),
    )(adj, st, obj_h0, bo0.reshape(1, D), Wq, Wvo)

    BR = 2048
    dinv, hp0 = pl.pallas_call(
        _dep_prep0_body,
        grid=(N_DEP_P // BR,),
        out_shape=(jax.ShapeDtypeStruct((N_DEP_P, 1), f32),
                   jax.ShapeDtypeStruct((N_DEP_P, D), f32)),
        in_specs=[pl.BlockSpec((BR, D), lambda i: (i, 0)),
                  pl.BlockSpec((D, D), lambda i: (0, 0)),
                  pl.BlockSpec((BR, 1), lambda i: (i, 0))],
        out_specs=(pl.BlockSpec((BR, 1), lambda i: (i, 0)),
                   pl.BlockSpec((BR, D), lambda i: (i, 0))),
    )(dep_x, Wt0, hist2)

    acc0 = _sc_aggregate(hp0, zeros_row, dsrc_a, ddst_a)

    dep_hb, obj_hb = pl.pallas_call(
        _biatt_body,
        grid=(BGRID,),
        out_shape=(jax.ShapeDtypeStruct((N_DEP, D), f32),
                   jax.ShapeDtypeStruct((N_OBJ, D), f32)),
        in_specs=[pl.BlockSpec((BT, D), lambda i: (i, 0)),
                  pl.BlockSpec((BT, D), lambda i: (i, 0)),
                  pl.BlockSpec((BT, D), lambda i: (i, 0)),
                  pl.BlockSpec((BT, 1), lambda i: (i, 0)),
                  pl.BlockSpec((1, D), lambda i: (0, 0)),
                  pl.BlockSpec((NH, D, D), lambda i: (0, 0, 0)),
                  pl.BlockSpec((NH, D, D), lambda i: (0, 0, 0)),
                  pl.BlockSpec((NH, N_OBJ, D), lambda i: (0, 0, 0)),
                  pl.BlockSpec((NH, N_OBJ, D), lambda i: (0, 0, 0))],
        out_specs=(pl.BlockSpec((BT, D), lambda i: (i, 0)),
                   pl.BlockSpec((N_OBJ, D), lambda i: (0, 0))),
        scratch_shapes=[pltpu.VMEM((NH, 1, N_OBJ), f32),
                        pltpu.VMEM((NH, 1, N_OBJ), f32),
                        pltpu.VMEM((NH, N_OBJ, D), f32)],
    )(acc0[0], acc0[1], hp0, dinv, bt0.reshape(1, D), Wk, Wvd, q, vo)

    hp1 = pl.pallas_call(
        _dep_prep1_body,
        grid=(N_DEP_P // BR,),
        out_shape=jax.ShapeDtypeStruct((N_DEP_P, D), f32),
        in_specs=[pl.BlockSpec((BR, D), lambda i: (i, 0)),
                  pl.BlockSpec((D, D), lambda i: (0, 0)),
                  pl.BlockSpec((BR, 1), lambda i: (i, 0))],
        out_specs=pl.BlockSpec((BR, D), lambda i: (i, 0)),
    )(dep_hb, Wt1, dinv)

    acc1 = _sc_aggregate(hp1, zeros_row, dsrc_a, ddst_a)

    hp2 = pl.pallas_call(
        _dep_prep2_body,
        grid=(N_DEP_P // BR,),
        out_shape=jax.ShapeDtypeStruct((N_DEP_P, D), f32),
        in_specs=[pl.BlockSpec((BR, D), lambda i: (i, 0)),
                  pl.BlockSpec((BR, D), lambda i: (i, 0)),
                  pl.BlockSpec((BR, D), lambda i: (i, 0)),
                  pl.BlockSpec((BR, 1), lambda i: (i, 0)),
                  pl.BlockSpec((1, D), lambda i: (0, 0)),
                  pl.BlockSpec((D, D), lambda i: (0, 0))],
        out_specs=pl.BlockSpec((BR, D), lambda i: (i, 0)),
    )(acc1[0], acc1[1], hp1, dinv, bt1.reshape(1, D), Wt2)

    acc2 = _sc_aggregate(hp2, zeros_row, dsrc_a, ddst_a)

    hid = F1.shape[1]
    a_cls = F2.shape[1]
    out = pl.pallas_call(
        _final_body,
        out_shape=jax.ShapeDtypeStruct((64, a_cls), f32),
        in_specs=[_full_spec((N_DEP, D)), _full_spec((N_DEP, D)),
                  _full_spec((N_DEP, D)), _full_spec((N_DEP, 1)),
                  _full_spec((1, D)), _full_spec((N_OBJ, D)),
                  _full_spec((N_DEP, 1)), _full_spec((N_OBJ, 1)),
                  _full_spec((2 * D, hid)), _full_spec((1, hid)),
                  _full_spec((hid, a_cls)), _full_spec((1, a_cls))],
        out_specs=_full_spec((64, a_cls)),
        compiler_params=pltpu.CompilerParams(
            vmem_limit_bytes=100 * 1024 * 1024),
    )(acc2[0, :N_DEP], acc2[1, :N_DEP], hp2[:N_DEP], dinv[:N_DEP],
      bt2.reshape(1, D), obj_hb,
      dep_batch.astype(i32).reshape(N_DEP, 1),
      obj_batch.astype(i32).reshape(N_OBJ, 1),
      F1, fb1.reshape(1, hid), F2, fb2.reshape(1, a_cls))
    return out


# speedup vs baseline: 1.2967x; 1.0076x over previous
import functools

import jax
import jax.numpy as jnp
from jax import lax
from jax.experimental import pallas as pl
from jax.experimental.pallas import tpu as pltpu
from jax.experimental.pallas import tpu_sc as plsc

N_DEP = 10000
N_OBJ = 1024
E_DEP = 320000
E_OBJ = 16384
D = 128
NH = 3

NSC = 2
NT = 16
KW = 128
JUNK = N_DEP
N_DEP_P = 10240
RPT = N_DEP_P // NT

W_AGG = 79
EPC = E_DEP // (NSC * NT)
W_HIST = 160
EPH = E_DEP // NT
W_OBJ = 8

_mesh = plsc.VectorSubcoreMesh(core_axis_name="c", subcore_axis_name="s")


@functools.partial(
    pl.kernel,
    out_type=jax.ShapeDtypeStruct((N_DEP_P,), jnp.float32),
    mesh=_mesh,
    scratch_types=[
        pltpu.VMEM_SHARED((N_DEP_P,), jnp.float32),
        pltpu.VMEM((W_HIST, KW), jnp.int32),
        pltpu.VMEM((KW,), jnp.float32),
    ],
)
def _sc_hist(ddst_hbm, zeros_hbm, hist_hbm, hist_sh, ddst_vm, ones_vm):
    c = lax.axis_index("c")
    s = lax.axis_index("s")
    CH = N_DEP_P // NT

    def fill_ones(j, _):
        ones_vm[pl.ds(j * 16, 16)] = jnp.full((16,), 1.0, jnp.float32)
        return 0
    lax.fori_loop(0, KW // 16, fill_ones, 0)

    @pl.when(c == 0)
    def _():
        pltpu.sync_copy(zeros_hbm.at[pl.ds(0, CH)],
                        hist_sh.at[pl.ds(s * CH, CH)])
        pltpu.sync_copy(ddst_hbm.at[s], ddst_vm)

    plsc.subcore_barrier()

    @pl.when(c == 0)
    def _():
        def body(w, _):
            pltpu.sync_copy(ones_vm, hist_sh.at[ddst_vm.at[w]], add=True)
            return 0
        lax.fori_loop(0, W_HIST, body, 0)

    plsc.subcore_barrier()

    @pl.when(c == 0)
    def _():
        pltpu.sync_copy(hist_sh.at[pl.ds(s * CH, CH)],
                        hist_hbm.at[pl.ds(s * CH, CH)])


@functools.partial(
    pl.kernel,
    out_type=jax.ShapeDtypeStruct((N_OBJ * N_OBJ,), jnp.float32),
    mesh=_mesh,
    scratch_types=[
        pltpu.VMEM_SHARED((N_OBJ * N_OBJ,), jnp.float32),
        pltpu.VMEM((W_OBJ, KW), jnp.int32),
        pltpu.VMEM((W_OBJ, KW), jnp.int32),
        pltpu.VMEM((KW,), jnp.int32),
        pltpu.VMEM((KW,), jnp.float32),
    ],
)
def _sc_adj(osrc_hbm, odst_hbm, zeros_hbm, adj_hbm,
            adj_sh, osrc_vm, odst_vm, flat_vm, ones_vm):
    c = lax.axis_index("c")
    s = lax.axis_index("s")
    MCH = (N_OBJ * N_OBJ) // NT

    def fill_ones(j, _):
        ones_vm[pl.ds(j * 16, 16)] = jnp.full((16,), 1.0, jnp.float32)
        return 0
    lax.fori_loop(0, KW // 16, fill_ones, 0)

    @pl.when(c == 0)
    def _():
        pltpu.sync_copy(zeros_hbm.at[pl.ds(0, MCH)],
                        adj_sh.at[pl.ds(s * MCH, MCH)])
        pltpu.sync_copy(osrc_hbm.at[s], osrc_vm)
        pltpu.sync_copy(odst_hbm.at[s], odst_vm)

    plsc.subcore_barrier()

    @pl.when(c == 0)
    def _():
        def body(w, _):
            def pack(j, _):
                sv = osrc_vm[w, pl.ds(j * 16, 16)]
                dv = odst_vm[w, pl.ds(j * 16, 16)]
                flat_vm[pl.ds(j * 16, 16)] = dv * N_OBJ + sv
                return 0
            lax.fori_loop(0, KW // 16, pack, 0)
            pltpu.sync_copy(ones_vm, adj_sh.at[flat_vm], add=True)
            return 0
        lax.fori_loop(0, W_OBJ, body, 0)

    plsc.subcore_barrier()

    @pl.when(c == 0)
    def _():
        pltpu.sync_copy(adj_sh.at[pl.ds(s * MCH, MCH)],
                        adj_hbm.at[pl.ds(s * MCH, MCH)])


@functools.partial(
    pl.kernel,
    out_type=jax.ShapeDtypeStruct((NSC, N_DEP_P, D), jnp.float32),
    mesh=_mesh,
    scratch_types=[
        pltpu.VMEM_SHARED((N_DEP_P, D), jnp.float32),
        pltpu.VMEM((W_AGG, KW), jnp.int32),
        pltpu.VMEM((W_AGG, KW), jnp.int32),
        pltpu.VMEM((KW, D), jnp.float32),
        pltpu.SemaphoreType.DMA,
    ],
)
def _sc_aggregate(hp_hbm, zeros_hbm, src_hbm, dst_hbm, acc_hbm,
                  acc_sh, src_vm, dst_vm, rows_vm, sem):
    c = lax.axis_index("c")
    s = lax.axis_index("s")
    r0 = s * RPT
    pltpu.sync_copy(zeros_hbm, acc_sh.at[pl.ds(r0, RPT)])
    pltpu.sync_copy(src_hbm.at[c, s], src_vm)
    pltpu.sync_copy(dst_hbm.at[c, s], dst_vm)
    plsc.subcore_barrier()

    def body(w, _):
        pltpu.async_copy(hp_hbm.at[src_vm.at[w]], rows_vm, sem).wait()
        pltpu.sync_copy(rows_vm, acc_sh.at[dst_vm.at[w]], add=True)
        return 0
    lax.fori_loop(0, W_AGG, body, 0)

    plsc.subcore_barrier()
    pltpu.sync_copy(acc_sh.at[pl.ds(r0, RPT)], acc_hbm.at[c, pl.ds(r0, RPT)])


def _obj_front_body(labels_ref, boxes_ref, ws_ref, wo_ref, wbs_ref, wbo_ref,
                    wo0_ref, st_ref, h0_ref):
    labels = labels_ref[...]
    ids = lax.broadcasted_iota(jnp.int32, (N_OBJ, 1600), 1)
    onehot = (ids == labels).astype(jnp.bfloat16)
    f32 = jnp.float32

    def ohdot(w):
        return lax.dot_general(onehot, w.astype(jnp.bfloat16),
                               (((1,), (0,)), ((), ())),
                               preferred_element_type=f32)
    subj = ohdot(ws_ref[...]) + boxes_ref[...] @ wbs_ref[...]
    objf = ohdot(wo_ref[...]) + boxes_ref[...] @ wbo_ref[...]
    st_ref[...] = lax.dot_general(objf, subj, (((1,), (1,)), ((), ())))
    h0_ref[...] = onehot.astype(f32) @ wo0_ref[...]


def _obj_conv_body(adj_ref, st_ref, h0_ref, bo0_ref, wq_ref, wvo_ref,
                   objh_ref, q_ref, vo_ref):
    sig = 1.0 / (1.0 + jnp.exp(-st_ref[...]))
    a = adj_ref[...] * sig
    deg = jnp.sum(a, axis=1, keepdims=True) + 1.0
    dinv = lax.rsqrt(jnp.maximum(deg, 1e-12))
    hp = dinv * h0_ref[...]
    out = dinv * (a @ hp + hp) + bo0_ref[...]
    objh_ref[...] = out
    for h in range(NH):
        q_ref[h] = out @ wq_ref[h]
        vo_ref[h] = out @ wvo_ref[h]


def _dep_prep0_body(x_ref, w_ref, hist_ref, dinv_ref, hp_ref):
    deg = hist_ref[...] + 1.0
    dinv = lax.rsqrt(jnp.maximum(deg, 1e-12))
    dinv_ref[...] = dinv
    hp_ref[...] = dinv * (x_ref[...] @ w_ref[...])


def _dep_prep2_body(acca_ref, accb_ref, hp1_ref, dinv_ref, bt1_ref, w_ref,
                    hp_ref):
    dinv = dinv_ref[...]
    dep1 = dinv * (acca_ref[...] + accb_ref[...] + hp1_ref[...]) + bt1_ref[...]
    hp_ref[...] = dinv * (dep1 @ w_ref[...])


BT = 400
BGRID = N_DEP // BT


def _biatt_body(acca_ref, accb_ref, hp0_ref, dinv_ref, bt0_ref, wk_ref,
                wvd_ref, q_ref, vo_ref, wt1_ref, hp1_ref, objo_ref,
                cmax_ref, csum_ref, cacc_ref):
    i = pl.program_id(0)
    scale = 1.0 / jnp.sqrt(jnp.float32(D))
    dinv = dinv_ref[...]
    dep_in = dinv * (acca_ref[...] + accb_ref[...] + hp0_ref[...]) \
        + bt0_ref[...]

    @pl.when(i == 0)
    def _():
        cmax_ref[...] = jnp.full_like(cmax_ref, -1e30)
        csum_ref[...] = jnp.zeros_like(csum_ref)
        cacc_ref[...] = jnp.zeros_like(cacc_ref)

    dep_acc = jnp.zeros((BT, D), jnp.float32)
    for h in range(NH):
        k_t = dep_in @ wk_ref[h]
        vd_t = dep_in @ wvd_ref[h]
        a = lax.dot_general(k_t, q_ref[h],
                            (((1,), (1,)), ((), ()))) * scale
        rmax = jnp.max(a, axis=1, keepdims=True)
        p = jnp.exp(a - rmax)
        rsum = jnp.sum(p, axis=1, keepdims=True)
        pn = (p / rsum).astype(jnp.bfloat16)
        dep_acc = dep_acc + lax.dot_general(
            pn, vo_ref[h].astype(jnp.bfloat16), (((1,), (0,)), ((), ())),
            preferred_element_type=jnp.float32)
        tmax = jnp.max(a, axis=0, keepdims=True)
        old_m = cmax_ref[h]
        new_m = jnp.maximum(old_m, tmax)
        corr = jnp.exp(old_m - new_m)
        e = jnp.exp(a - new_m)
        cmax_ref[h] = new_m
        csum_ref[h] = csum_ref[h] * corr + jnp.sum(e, axis=0, keepdims=True)
        corr_t = corr.reshape(N_OBJ, 1)
        cacc_ref[h] = cacc_ref[h] * corr_t + lax.dot_general(
            e.astype(jnp.bfloat16), vd_t.astype(jnp.bfloat16),
            (((0,), (0,)), ((), ())),
            preferred_element_type=jnp.float32)

    hp1_ref[...] = dinv * ((dep_acc * (1.0 / NH)) @ wt1_ref[...])

    @pl.when(i == BGRID - 1)
    def _():
        acc = jnp.zeros((N_OBJ, D), jnp.float32)
        for h in range(NH):
            acc = acc + cacc_ref[h] / csum_ref[h].reshape(N_OBJ, 1)
        objo_ref[...] = acc * (1.0 / NH)


def _final_body(acca_ref, accb_ref, hp2_ref, dinv_ref, bt2_ref, objh_ref,
                dbatch_ref, obatch_ref, f1_ref, fb1_ref, f2_ref, fb2_ref,
                out_ref):
    dinv = dinv_ref[...]
    dep2 = dinv * (acca_ref[...] + accb_ref[...] + hp2_ref[...]) \
        + bt2_ref[...]
    db = dbatch_ref[...]
    ids = lax.broadcasted_iota(jnp.int32, (N_DEP, 64), 1)
    ohd = (ids == db).astype(jnp.float32)
    dsum = lax.dot_general(ohd, dep2, (((0,), (0,)), ((), ())))
    dcnt = jnp.sum(ohd, axis=0, keepdims=True).reshape(64, 1)
    dep_p = dsum / jnp.maximum(dcnt, 1.0)

    ob = obatch_ref[...]
    ids_o = lax.broadcasted_iota(jnp.int32, (N_OBJ, 64), 1)
    oho = (ids_o == ob).astype(jnp.float32)
    osum = lax.dot_general(oho, objh_ref[...], (((0,), (0,)), ((), ())))
    ocnt = jnp.sum(oho, axis=0, keepdims=True).reshape(64, 1)
    obj_p = osum / jnp.maximum(ocnt, 1.0)

    fused = jnp.concatenate([dep_p, obj_p], axis=1)
    hmid = fused @ f1_ref[...] + fb1_ref[...]
    logits = hmid @ f2_ref[...] + fb2_ref[...]
    lmax = jnp.max(logits, axis=1, keepdims=True)
    lse = jnp.log(jnp.sum(jnp.exp(logits - lmax), axis=1, keepdims=True)) + lmax
    out_ref[...] = logits - lse


def _full_spec(shape):
    return pl.BlockSpec(shape, lambda *_: tuple(0 for _ in shape))


def kernel(dep_x, dep_edge_index, dep_batch, obj_boxes, obj_labels,
           obj_edge_index, obj_batch, Wt0, bt0, Wo0, bo0, Wk, Wq, Wvd, Wvo,
           Wt1, bt1, Wt2, bt2, Ws_rel, Wo_rel, Wbs, Wbo, F1, fb1, F2, fb2):
    f32 = jnp.float32
    i32 = jnp.int32

    padc = W_AGG * KW - EPC
    dsrc_a = dep_edge_index[0].astype(i32).reshape(NSC * NT, EPC)
    dsrc_a = jnp.concatenate(
        [dsrc_a, jnp.zeros((NSC * NT, padc), i32)], axis=1)
    dsrc_a = dsrc_a.reshape(NSC, NT, W_AGG, KW)
    ddst_a = dep_edge_index[1].astype(i32).reshape(NSC * NT, EPC)
    ddst_a = jnp.concatenate(
        [ddst_a, jnp.full((NSC * NT, padc), JUNK, i32)], axis=1)
    ddst_a = ddst_a.reshape(NSC, NT, W_AGG, KW)
    padh = W_HIST * KW - EPH
    ddst_h = dep_edge_index[1].astype(i32).reshape(NT, EPH)
    ddst_h = jnp.concatenate(
        [ddst_h, jnp.full((NT, padh), JUNK, i32)], axis=1)
    ddst_h = ddst_h.reshape(NT, W_HIST, KW)
    osrc = obj_edge_index[0].astype(i32).reshape(NT, W_OBJ, KW)
    odst = obj_edge_index[1].astype(i32).reshape(NT, W_OBJ, KW)
    zeros_row = jnp.zeros((RPT, D), f32)
    zeros_hist = jnp.zeros((N_DEP_P // NT,), f32)
    zeros_adj = jnp.zeros(((N_OBJ * N_OBJ) // NT,), f32)

    hist = _sc_hist(ddst_h, zeros_hist)
    adj = _sc_adj(osrc, odst, zeros_adj).reshape(N_OBJ, N_OBJ)
    hist2 = hist.reshape(N_DEP_P, 1)

    st, obj_h0 = pl.pallas_call(
        _obj_front_body,
        out_shape=(jax.ShapeDtypeStruct((N_OBJ, N_OBJ), f32),
                   jax.ShapeDtypeStruct((N_OBJ, D), f32)),
        in_specs=[_full_spec((N_OBJ, 1)), _full_spec((N_OBJ, 4)),
                  _full_spec((1600, 64)), _full_spec((1600, 64)),
                  _full_spec((4, 64)), _full_spec((4, 64)),
                  _full_spec((1600, D))],
        out_specs=(_full_spec((N_OBJ, N_OBJ)), _full_spec((N_OBJ, D))),
    )(obj_labels.astype(i32).reshape(N_OBJ, 1), obj_boxes, Ws_rel, Wo_rel,
      Wbs, Wbo, Wo0)

    obj_h, q, vo = pl.pallas_call(
        _obj_conv_body,
        out_shape=(jax.ShapeDtypeStruct((N_OBJ, D), f32),
                   jax.ShapeDtypeStruct((NH, N_OBJ, D), f32),
                   jax.ShapeDtypeStruct((NH, N_OBJ, D), f32)),
        in_specs=[_full_spec((N_OBJ, N_OBJ)), _full_spec((N_OBJ, N_OBJ)),
                  _full_spec((N_OBJ, D)), _full_spec((1, D)),
                  _full_spec((NH, D, D)), _full_spec((NH, D, D))],
        out_specs=(_full_spec((N_OBJ, D)), _full_spec((NH, N_OBJ, D)),
                   _full_spec((NH, N_OBJ, D))),
    )(adj, st, obj_h0, bo0.reshape(1, D), Wq, Wvo)

    BR = 2048
    dinv, hp0 = pl.pallas_call(
        _dep_prep0_body,
        grid=(N_DEP_P // BR,),
        out_shape=(jax.ShapeDtypeStruct((N_DEP_P, 1), f32),
                   jax.ShapeDtypeStruct((N_DEP_P, D), f32)),
        in_specs=[pl.BlockSpec((BR, D), lambda i: (i, 0)),
                  pl.BlockSpec((D, D), lambda i: (0, 0)),
                  pl.BlockSpec((BR, 1), lambda i: (i, 0))],
        out_specs=(pl.BlockSpec((BR, 1), lambda i: (i, 0)),
                   pl.BlockSpec((BR, D), lambda i: (i, 0))),
    )(dep_x, Wt0, hist2)

    acc0 = _sc_aggregate(hp0, zeros_row, dsrc_a, ddst_a)

    hp1, obj_hb = pl.pallas_call(
        _biatt_body,
        grid=(BGRID,),
        out_shape=(jax.ShapeDtypeStruct((N_DEP, D), f32),
                   jax.ShapeDtypeStruct((N_OBJ, D), f32)),
        in_specs=[pl.BlockSpec((BT, D), lambda i: (i, 0)),
                  pl.BlockSpec((BT, D), lambda i: (i, 0)),
                  pl.BlockSpec((BT, D), lambda i: (i, 0)),
                  pl.BlockSpec((BT, 1), lambda i: (i, 0)),
                  pl.BlockSpec((1, D), lambda i: (0, 0)),
                  pl.BlockSpec((NH, D, D), lambda i: (0, 0, 0)),
                  pl.BlockSpec((NH, D, D), lambda i: (0, 0, 0)),
                  pl.BlockSpec((NH, N_OBJ, D), lambda i: (0, 0, 0)),
                  pl.BlockSpec((NH, N_OBJ, D), lambda i: (0, 0, 0)),
                  pl.BlockSpec((D, D), lambda i: (0, 0))],
        out_specs=(pl.BlockSpec((BT, D), lambda i: (i, 0)),
                   pl.BlockSpec((N_OBJ, D), lambda i: (0, 0))),
        scratch_shapes=[pltpu.VMEM((NH, 1, N_OBJ), f32),
                        pltpu.VMEM((NH, 1, N_OBJ), f32),
                        pltpu.VMEM((NH, N_OBJ, D), f32)],
    )(acc0[0], acc0[1], hp0, dinv, bt0.reshape(1, D), Wk, Wvd, q, vo, Wt1)

    acc1 = _sc_aggregate(hp1, zeros_row, dsrc_a, ddst_a)

    hp2 = pl.pallas_call(
        _dep_prep2_body,
        grid=(N_DEP_P // BR,),
        out_shape=jax.ShapeDtypeStruct((N_DEP_P, D), f32),
        in_specs=[pl.BlockSpec((BR, D), lambda i: (i, 0)),
                  pl.BlockSpec((BR, D), lambda i: (i, 0)),
                  pl.BlockSpec((BR, D), lambda i: (i, 0)),
                  pl.BlockSpec((BR, 1), lambda i: (i, 0)),
                  pl.BlockSpec((1, D), lambda i: (0, 0)),
                  pl.BlockSpec((D, D), lambda i: (0, 0))],
        out_specs=pl.BlockSpec((BR, D), lambda i: (i, 0)),
    )(acc1[0], acc1[1], hp1, dinv, bt1.reshape(1, D), Wt2)

    acc2 = _sc_aggregate(hp2, zeros_row, dsrc_a, ddst_a)

    hid = F1.shape[1]
    a_cls = F2.shape[1]
    out = pl.pallas_call(
        _final_body,
        out_shape=jax.ShapeDtypeStruct((64, a_cls), f32),
        in_specs=[_full_spec((N_DEP, D)), _full_spec((N_DEP, D)),
                  _full_spec((N_DEP, D)), _full_spec((N_DEP, 1)),
                  _full_spec((1, D)), _full_spec((N_OBJ, D)),
                  _full_spec((N_DEP, 1)), _full_spec((N_OBJ, 1)),
                  _full_spec((2 * D, hid)), _full_spec((1, hid)),
                  _full_spec((hid, a_cls)), _full_spec((1, a_cls))],
        out_specs=_full_spec((64, a_cls)),
        compiler_params=pltpu.CompilerParams(
            vmem_limit_bytes=100 * 1024 * 1024),
    )(acc2[0, :N_DEP], acc2[1, :N_DEP], hp2[:N_DEP], dinv[:N_DEP],
      bt2.reshape(1, D), obj_hb,
      dep_batch.astype(i32).reshape(N_DEP, 1),
      obj_batch.astype(i32).reshape(N_OBJ, 1),
      F1, fb1.reshape(1, hid), F2, fb2.reshape(1, a_cls))
    return out


# speedup vs baseline: 1.3164x; 1.0152x over previous
import functools

import jax
import jax.numpy as jnp
from jax import lax
from jax.experimental import pallas as pl
from jax.experimental.pallas import tpu as pltpu
from jax.experimental.pallas import tpu_sc as plsc

N_DEP = 10000
N_OBJ = 1024
E_DEP = 320000
E_OBJ = 16384
D = 128
NH = 3

NSC = 2
NT = 16
KW = 128
JUNK = N_DEP
N_DEP_P = 10240
RPT = N_DEP_P // NT

W_AGG = 79
EPC = E_DEP // (NSC * NT)
W_OBJ = 8

_mesh = plsc.VectorSubcoreMesh(core_axis_name="c", subcore_axis_name="s")


@functools.partial(
    pl.kernel,
    out_type=jax.ShapeDtypeStruct((NSC, N_DEP_P), jnp.float32),
    mesh=_mesh,
    scratch_types=[
        pltpu.VMEM_SHARED((N_DEP_P,), jnp.float32),
        pltpu.VMEM((W_AGG, KW), jnp.int32),
        pltpu.VMEM((KW,), jnp.float32),
    ],
)
def _sc_hist(ddst_hbm, zeros_hbm, hist_hbm, hist_sh, ddst_vm, ones_vm):
    c = lax.axis_index("c")
    s = lax.axis_index("s")
    CH = N_DEP_P // NT

    def fill_ones(j, _):
        ones_vm[pl.ds(j * 16, 16)] = jnp.full((16,), 1.0, jnp.float32)
        return 0
    lax.fori_loop(0, KW // 16, fill_ones, 0)

    pltpu.sync_copy(zeros_hbm.at[pl.ds(0, CH)],
                    hist_sh.at[pl.ds(s * CH, CH)])
    pltpu.sync_copy(ddst_hbm.at[c, s], ddst_vm)
    plsc.subcore_barrier()

    def body(w, _):
        pltpu.sync_copy(ones_vm, hist_sh.at[ddst_vm.at[w]], add=True)
        return 0
    lax.fori_loop(0, W_AGG, body, 0)

    plsc.subcore_barrier()
    pltpu.sync_copy(hist_sh.at[pl.ds(s * CH, CH)],
                    hist_hbm.at[c, pl.ds(s * CH, CH)])


@functools.partial(
    pl.kernel,
    out_type=jax.ShapeDtypeStruct((N_OBJ * N_OBJ,), jnp.float32),
    mesh=_mesh,
    scratch_types=[
        pltpu.VMEM_SHARED((N_OBJ * N_OBJ,), jnp.float32),
        pltpu.VMEM((W_OBJ, KW), jnp.int32),
        pltpu.VMEM((W_OBJ, KW), jnp.int32),
        pltpu.VMEM((KW,), jnp.int32),
        pltpu.VMEM((KW,), jnp.float32),
    ],
)
def _sc_adj(osrc_hbm, odst_hbm, zeros_hbm, adj_hbm,
            adj_sh, osrc_vm, odst_vm, flat_vm, ones_vm):
    c = lax.axis_index("c")
    s = lax.axis_index("s")
    MCH = (N_OBJ * N_OBJ) // NT

    def fill_ones(j, _):
        ones_vm[pl.ds(j * 16, 16)] = jnp.full((16,), 1.0, jnp.float32)
        return 0
    lax.fori_loop(0, KW // 16, fill_ones, 0)

    @pl.when(c == 0)
    def _():
        pltpu.sync_copy(zeros_hbm.at[pl.ds(0, MCH)],
                        adj_sh.at[pl.ds(s * MCH, MCH)])
        pltpu.sync_copy(osrc_hbm.at[s], osrc_vm)
        pltpu.sync_copy(odst_hbm.at[s], odst_vm)

    plsc.subcore_barrier()

    @pl.when(c == 0)
    def _():
        def body(w, _):
            def pack(j, _):
                sv = osrc_vm[w, pl.ds(j * 16, 16)]
                dv = odst_vm[w, pl.ds(j * 16, 16)]
                flat_vm[pl.ds(j * 16, 16)] = dv * N_OBJ + sv
                return 0
            lax.fori_loop(0, KW // 16, pack, 0)
            pltpu.sync_copy(ones_vm, adj_sh.at[flat_vm], add=True)
            return 0
        lax.fori_loop(0, W_OBJ, body, 0)

    plsc.subcore_barrier()

    @pl.when(c == 0)
    def _():
        pltpu.sync_copy(adj_sh.at[pl.ds(s * MCH, MCH)],
                        adj_hbm.at[pl.ds(s * MCH, MCH)])


@functools.partial(
    pl.kernel,
    out_type=jax.ShapeDtypeStruct((NSC, N_DEP_P, D), jnp.float32),
    mesh=_mesh,
    scratch_types=[
        pltpu.VMEM_SHARED((N_DEP_P, D), jnp.float32),
        pltpu.VMEM((W_AGG, KW), jnp.int32),
        pltpu.VMEM((W_AGG, KW), jnp.int32),
        pltpu.VMEM((KW, D), jnp.float32),
        pltpu.SemaphoreType.DMA,
    ],
)
def _sc_aggregate(hp_hbm, zeros_hbm, src_hbm, dst_hbm, acc_hbm,
                  acc_sh, src_vm, dst_vm, rows_vm, sem):
    c = lax.axis_index("c")
    s = lax.axis_index("s")
    r0 = s * RPT
    pltpu.sync_copy(zeros_hbm, acc_sh.at[pl.ds(r0, RPT)])
    pltpu.sync_copy(src_hbm.at[c, s], src_vm)
    pltpu.sync_copy(dst_hbm.at[c, s], dst_vm)
    plsc.subcore_barrier()

    def body(w, _):
        pltpu.async_copy(hp_hbm.at[src_vm.at[w]], rows_vm, sem).wait()
        pltpu.sync_copy(rows_vm, acc_sh.at[dst_vm.at[w]], add=True)
        return 0
    lax.fori_loop(0, W_AGG, body, 0)

    plsc.subcore_barrier()
    pltpu.sync_copy(acc_sh.at[pl.ds(r0, RPT)], acc_hbm.at[c, pl.ds(r0, RPT)])


def _obj_front_body(labels_ref, boxes_ref, ws_ref, wo_ref, wbs_ref, wbo_ref,
                    wo0_ref, st_ref, h0_ref):
    labels = labels_ref[...]
    ids = lax.broadcasted_iota(jnp.int32, (N_OBJ, 1600), 1)
    onehot = (ids == labels).astype(jnp.bfloat16)
    f32 = jnp.float32

    def ohdot(w):
        return lax.dot_general(onehot, w.astype(jnp.bfloat16),
                               (((1,), (0,)), ((), ())),
                               preferred_element_type=f32)
    subj = ohdot(ws_ref[...]) + boxes_ref[...] @ wbs_ref[...]
    objf = ohdot(wo_ref[...]) + boxes_ref[...] @ wbo_ref[...]
    st_ref[...] = lax.dot_general(objf, subj, (((1,), (1,)), ((), ())))
    h0_ref[...] = onehot.astype(f32) @ wo0_ref[...]


def _obj_conv_body(adj_ref, st_ref, h0_ref, bo0_ref, wq_ref, wvo_ref,
                   objh_ref, q_ref, vo_ref):
    sig = 1.0 / (1.0 + jnp.exp(-st_ref[...]))
    a = adj_ref[...] * sig
    deg = jnp.sum(a, axis=1, keepdims=True) + 1.0
    dinv = lax.rsqrt(jnp.maximum(deg, 1e-12))
    hp = dinv * h0_ref[...]
    out = dinv * (a @ hp + hp) + bo0_ref[...]
    objh_ref[...] = out
    for h in range(NH):
        q_ref[h] = out @ wq_ref[h]
        vo_ref[h] = out @ wvo_ref[h]


def _dep_prep0_body(x_ref, w_ref, hist0_ref, hist1_ref, dinv_ref, hp_ref):
    deg = hist0_ref[...] + hist1_ref[...] + 1.0
    dinv = lax.rsqrt(jnp.maximum(deg, 1e-12))
    dinv_ref[...] = dinv
    hp_ref[...] = dinv * (x_ref[...] @ w_ref[...])


def _dep_prep2_body(acca_ref, accb_ref, hp1_ref, dinv_ref, bt1_ref, w_ref,
                    hp_ref):
    dinv = dinv_ref[...]
    dep1 = dinv * (acca_ref[...] + accb_ref[...] + hp1_ref[...]) + bt1_ref[...]
    hp_ref[...] = dinv * (dep1 @ w_ref[...])


BT = 400
BGRID = N_DEP // BT


def _biatt_body(acca_ref, accb_ref, hp0_ref, dinv_ref, bt0_ref, wk_ref,
                wvd_ref, q_ref, vo_ref, wt1_ref, hp1_ref, objo_ref,
                cmax_ref, csum_ref, cacc_ref):
    i = pl.program_id(0)
    scale = 1.0 / jnp.sqrt(jnp.float32(D))
    dinv = dinv_ref[...]
    dep_in = dinv * (acca_ref[...] + accb_ref[...] + hp0_ref[...]) \
        + bt0_ref[...]

    @pl.when(i == 0)
    def _():
        cmax_ref[...] = jnp.full_like(cmax_ref, -1e30)
        csum_ref[...] = jnp.zeros_like(csum_ref)
        cacc_ref[...] = jnp.zeros_like(cacc_ref)

    dep_acc = jnp.zeros((BT, D), jnp.float32)
    for h in range(NH):
        k_t = dep_in @ wk_ref[h]
        vd_t = dep_in @ wvd_ref[h]
        a = lax.dot_general(k_t, q_ref[h],
                            (((1,), (1,)), ((), ()))) * scale
        rmax = jnp.max(a, axis=1, keepdims=True)
        p = jnp.exp(a - rmax)
        rsum = jnp.sum(p, axis=1, keepdims=True)
        pn = (p / rsum).astype(jnp.bfloat16)
        dep_acc = dep_acc + lax.dot_general(
            pn, vo_ref[h].astype(jnp.bfloat16), (((1,), (0,)), ((), ())),
            preferred_element_type=jnp.float32)
        tmax = jnp.max(a, axis=0, keepdims=True)
        old_m = cmax_ref[h]
        new_m = jnp.maximum(old_m, tmax)
        corr = jnp.exp(old_m - new_m)
        e = jnp.exp(a - new_m)
        cmax_ref[h] = new_m
        csum_ref[h] = csum_ref[h] * corr + jnp.sum(e, axis=0, keepdims=True)
        corr_t = corr.reshape(N_OBJ, 1)
        cacc_ref[h] = cacc_ref[h] * corr_t + lax.dot_general(
            e.astype(jnp.bfloat16), vd_t.astype(jnp.bfloat16),
            (((0,), (0,)), ((), ())),
            preferred_element_type=jnp.float32)

    hp1_ref[...] = dinv * ((dep_acc * (1.0 / NH)) @ wt1_ref[...])

    @pl.when(i == BGRID - 1)
    def _():
        acc = jnp.zeros((N_OBJ, D), jnp.float32)
        for h in range(NH):
            acc = acc + cacc_ref[h] / csum_ref[h].reshape(N_OBJ, 1)
        objo_ref[...] = acc * (1.0 / NH)


def _final_body(acca_ref, accb_ref, hp2_ref, dinv_ref, bt2_ref, objh_ref,
                dbatch_ref, obatch_ref, f1_ref, fb1_ref, f2_ref, fb2_ref,
                out_ref):
    dinv = dinv_ref[...]
    dep2 = dinv * (acca_ref[...] + accb_ref[...] + hp2_ref[...]) \
        + bt2_ref[...]
    db = dbatch_ref[...]
    ids = lax.broadcasted_iota(jnp.int32, (N_DEP, 64), 1)
    ohd = (ids == db).astype(jnp.float32)
    dsum = lax.dot_general(ohd, dep2, (((0,), (0,)), ((), ())))
    dcnt = jnp.sum(ohd, axis=0, keepdims=True).reshape(64, 1)
    dep_p = dsum / jnp.maximum(dcnt, 1.0)

    ob = obatch_ref[...]
    ids_o = lax.broadcasted_iota(jnp.int32, (N_OBJ, 64), 1)
    oho = (ids_o == ob).astype(jnp.float32)
    osum = lax.dot_general(oho, objh_ref[...], (((0,), (0,)), ((), ())))
    ocnt = jnp.sum(oho, axis=0, keepdims=True).reshape(64, 1)
    obj_p = osum / jnp.maximum(ocnt, 1.0)

    fused = jnp.concatenate([dep_p, obj_p], axis=1)
    hmid = fused @ f1_ref[...] + fb1_ref[...]
    logits = hmid @ f2_ref[...] + fb2_ref[...]
    lmax = jnp.max(logits, axis=1, keepdims=True)
    lse = jnp.log(jnp.sum(jnp.exp(logits - lmax), axis=1, keepdims=True)) + lmax
    out_ref[...] = logits - lse


def _full_spec(shape):
    return pl.BlockSpec(shape, lambda *_: tuple(0 for _ in shape))


def kernel(dep_x, dep_edge_index, dep_batch, obj_boxes, obj_labels,
           obj_edge_index, obj_batch, Wt0, bt0, Wo0, bo0, Wk, Wq, Wvd, Wvo,
           Wt1, bt1, Wt2, bt2, Ws_rel, Wo_rel, Wbs, Wbo, F1, fb1, F2, fb2):
    f32 = jnp.float32
    i32 = jnp.int32

    padc = W_AGG * KW - EPC
    dsrc_a = dep_edge_index[0].astype(i32).reshape(NSC * NT, EPC)
    dsrc_a = jnp.concatenate(
        [dsrc_a, jnp.zeros((NSC * NT, padc), i32)], axis=1)
    dsrc_a = dsrc_a.reshape(NSC, NT, W_AGG, KW)
    ddst_a = dep_edge_index[1].astype(i32).reshape(NSC * NT, EPC)
    ddst_a = jnp.concatenate(
        [ddst_a, jnp.full((NSC * NT, padc), JUNK, i32)], axis=1)
    ddst_a = ddst_a.reshape(NSC, NT, W_AGG, KW)
    osrc = obj_edge_index[0].astype(i32).reshape(NT, W_OBJ, KW)
    odst = obj_edge_index[1].astype(i32).reshape(NT, W_OBJ, KW)
    zeros_row = jnp.zeros((RPT, D), f32)
    zeros_hist = jnp.zeros((N_DEP_P // NT,), f32)
    zeros_adj = jnp.zeros(((N_OBJ * N_OBJ) // NT,), f32)

    hist = _sc_hist(ddst_a, zeros_hist)
    adj = _sc_adj(osrc, odst, zeros_adj).reshape(N_OBJ, N_OBJ)
    hist0 = hist[0].reshape(N_DEP_P, 1)
    hist1 = hist[1].reshape(N_DEP_P, 1)

    st, obj_h0 = pl.pallas_call(
        _obj_front_body,
        out_shape=(jax.ShapeDtypeStruct((N_OBJ, N_OBJ), f32),
                   jax.ShapeDtypeStruct((N_OBJ, D), f32)),
        in_specs=[_full_spec((N_OBJ, 1)), _full_spec((N_OBJ, 4)),
                  _full_spec((1600, 64)), _full_spec((1600, 64)),
                  _full_spec((4, 64)), _full_spec((4, 64)),
                  _full_spec((1600, D))],
        out_specs=(_full_spec((N_OBJ, N_OBJ)), _full_spec((N_OBJ, D))),
    )(obj_labels.astype(i32).reshape(N_OBJ, 1), obj_boxes, Ws_rel, Wo_rel,
      Wbs, Wbo, Wo0)

    obj_h, q, vo = pl.pallas_call(
        _obj_conv_body,
        out_shape=(jax.ShapeDtypeStruct((N_OBJ, D), f32),
                   jax.ShapeDtypeStruct((NH, N_OBJ, D), f32),
                   jax.ShapeDtypeStruct((NH, N_OBJ, D), f32)),
        in_specs=[_full_spec((N_OBJ, N_OBJ)), _full_spec((N_OBJ, N_OBJ)),
                  _full_spec((N_OBJ, D)), _full_spec((1, D)),
                  _full_spec((NH, D, D)), _full_spec((NH, D, D))],
        out_specs=(_full_spec((N_OBJ, D)), _full_spec((NH, N_OBJ, D)),
                   _full_spec((NH, N_OBJ, D))),
    )(adj, st, obj_h0, bo0.reshape(1, D), Wq, Wvo)

    BR = 2048
    dinv, hp0 = pl.pallas_call(
        _dep_prep0_body,
        grid=(N_DEP_P // BR,),
        out_shape=(jax.ShapeDtypeStruct((N_DEP_P, 1), f32),
                   jax.ShapeDtypeStruct((N_DEP_P, D), f32)),
        in_specs=[pl.BlockSpec((BR, D), lambda i: (i, 0)),
                  pl.BlockSpec((D, D), lambda i: (0, 0)),
                  pl.BlockSpec((BR, 1), lambda i: (i, 0)),
                  pl.BlockSpec((BR, 1), lambda i: (i, 0))],
        out_specs=(pl.BlockSpec((BR, 1), lambda i: (i, 0)),
                   pl.BlockSpec((BR, D), lambda i: (i, 0))),
    )(dep_x, Wt0, hist0, hist1)

    acc0 = _sc_aggregate(hp0, zeros_row, dsrc_a, ddst_a)

    hp1, obj_hb = pl.pallas_call(
        _biatt_body,
        grid=(BGRID,),
        out_shape=(jax.ShapeDtypeStruct((N_DEP, D), f32),
                   jax.ShapeDtypeStruct((N_OBJ, D), f32)),
        in_specs=[pl.BlockSpec((BT, D), lambda i: (i, 0)),
                  pl.BlockSpec((BT, D), lambda i: (i, 0)),
                  pl.BlockSpec((BT, D), lambda i: (i, 0)),
                  pl.BlockSpec((BT, 1), lambda i: (i, 0)),
                  pl.BlockSpec((1, D), lambda i: (0, 0)),
                  pl.BlockSpec((NH, D, D), lambda i: (0, 0, 0)),
                  pl.BlockSpec((NH, D, D), lambda i: (0, 0, 0)),
                  pl.BlockSpec((NH, N_OBJ, D), lambda i: (0, 0, 0)),
                  pl.BlockSpec((NH, N_OBJ, D), lambda i: (0, 0, 0)),
                  pl.BlockSpec((D, D), lambda i: (0, 0))],
        out_specs=(pl.BlockSpec((BT, D), lambda i: (i, 0)),
                   pl.BlockSpec((N_OBJ, D), lambda i: (0, 0))),
        scratch_shapes=[pltpu.VMEM((NH, 1, N_OBJ), f32),
                        pltpu.VMEM((NH, 1, N_OBJ), f32),
                        pltpu.VMEM((NH, N_OBJ, D), f32)],
    )(acc0[0], acc0[1], hp0, dinv, bt0.reshape(1, D), Wk, Wvd, q, vo, Wt1)

    acc1 = _sc_aggregate(hp1, zeros_row, dsrc_a, ddst_a)

    hp2 = pl.pallas_call(
        _dep_prep2_body,
        grid=(N_DEP_P // BR,),
        out_shape=jax.ShapeDtypeStruct((N_DEP_P, D), f32),
        in_specs=[pl.BlockSpec((BR, D), lambda i: (i, 0)),
                  pl.BlockSpec((BR, D), lambda i: (i, 0)),
                  pl.BlockSpec((BR, D), lambda i: (i, 0)),
                  pl.BlockSpec((BR, 1), lambda i: (i, 0)),
                  pl.BlockSpec((1, D), lambda i: (0, 0)),
                  pl.BlockSpec((D, D), lambda i: (0, 0))],
        out_specs=pl.BlockSpec((BR, D), lambda i: (i, 0)),
    )(acc1[0], acc1[1], hp1, dinv, bt1.reshape(1, D), Wt2)

    acc2 = _sc_aggregate(hp2, zeros_row, dsrc_a, ddst_a)

    hid = F1.shape[1]
    a_cls = F2.shape[1]
    out = pl.pallas_call(
        _final_body,
        out_shape=jax.ShapeDtypeStruct((64, a_cls), f32),
        in_specs=[_full_spec((N_DEP, D)), _full_spec((N_DEP, D)),
                  _full_spec((N_DEP, D)), _full_spec((N_DEP, 1)),
                  _full_spec((1, D)), _full_spec((N_OBJ, D)),
                  _full_spec((N_DEP, 1)), _full_spec((N_OBJ, 1)),
                  _full_spec((2 * D, hid)), _full_spec((1, hid)),
                  _full_spec((hid, a_cls)), _full_spec((1, a_cls))],
        out_specs=_full_spec((64, a_cls)),
        compiler_params=pltpu.CompilerParams(
            vmem_limit_bytes=100 * 1024 * 1024),
    )(acc2[0, :N_DEP], acc2[1, :N_DEP], hp2[:N_DEP], dinv[:N_DEP],
      bt2.reshape(1, D), obj_hb,
      dep_batch.astype(i32).reshape(N_DEP, 1),
      obj_batch.astype(i32).reshape(N_OBJ, 1),
      F1, fb1.reshape(1, hid), F2, fb2.reshape(1, a_cls))
    return out
